# baseline (speedup 1.0000x reference)
import functools

import jax
import jax.numpy as jnp
from jax import lax
from jax.experimental import pallas as pl
from jax.experimental.pallas import tpu as pltpu
from jax.experimental.pallas import tpu_sc as plsc

_N = 10000
_M = 5000
_D = 128
_NP = 10112
_MP = 5120
_NCORE = 2
_NSUB = 16
_NTILE = _NCORE * _NSUB
_LANE = 16
_B = 128


def _ceil_to(v, m):
    return (v + m - 1) // m * m


def _mesh(nc=_NCORE):
    return plsc.VectorSubcoreMesh(core_axis_name="c", subcore_axis_name="s",
                                  num_cores=nc)


def _make_deg(nblk):
    nb_sub = nblk // _NSUB
    n_tile = _NP // _NSUB

    @functools.partial(
        pl.kernel,
        out_type=jax.ShapeDtypeStruct((_NCORE * _NP, _D), jnp.float32),
        mesh=_mesh(),
        scratch_types=[
            pltpu.VMEM((nb_sub, _B), jnp.int32),
            pltpu.VMEM((_B, _D), jnp.float32),
            pltpu.VMEM_SHARED((_NP, _D), jnp.float32),
        ],
        name="tricon_deg",
    )
    def deg(idx_hbm, out, idx_v, ones_v, hist):
        c = lax.axis_index("c")
        s = lax.axis_index("s")

        zero16 = jnp.zeros((_LANE,), jnp.float32)

        def zrow(j, _):
            for k in range(_D // _LANE):
                ones_v[j, pl.ds(k * _LANE, _LANE)] = zero16
            return 0

        lax.fori_loop(0, _B, zrow, 0)

        off = 0
        while off < n_tile:
            ch = min(_B, n_tile - off)
            pltpu.sync_copy(ones_v.at[pl.ds(0, ch)],
                            hist.at[pl.ds(s * n_tile + off, ch)])
            off += ch

        one16 = jnp.full((_LANE,), 1.0, jnp.float32)

        def orow(j, _):
            for k in range(_D // _LANE):
                ones_v[j, pl.ds(k * _LANE, _LANE)] = one16
            return 0

        lax.fori_loop(0, _B, orow, 0)
        plsc.subcore_barrier()

        pltpu.sync_copy(idx_hbm.at[pl.ds(c * nblk + s * nb_sub, nb_sub)],
                        idx_v)

        def body(j, _):
            pltpu.sync_copy(ones_v, hist.at[idx_v.at[j]], add=True)
            return 0

        lax.fori_loop(0, nb_sub, body, 0)
        plsc.subcore_barrier()

        off = 0
        while off < n_tile:
            ch = min(_B, n_tile - off)
            row0 = s * n_tile + off
            pltpu.sync_copy(hist.at[pl.ds(row0, ch)],
                            ones_v.at[pl.ds(0, ch)])
            pltpu.sync_copy(ones_v.at[pl.ds(0, ch)],
                            out.at[pl.ds(c * _NP + row0, ch)])
            off += ch

    return deg


def _make_spmm(r_pad, s_pad, nblk, name, f0, ncores=_NCORE):
    nb_pair = 160
    f1 = nb_pair - f0
    fmax = max(f0, f1)
    s_tile = s_pad // _NSUB

    @functools.partial(
        pl.kernel,
        out_type=jax.ShapeDtypeStruct((ncores * s_pad, _D), jnp.float32),
        mesh=_mesh(ncores),
        scratch_types=[
            pltpu.VMEM((8, _B), jnp.int32),
            pltpu.VMEM((8, _B), jnp.int32),
            pltpu.VMEM((_B, _D), jnp.float32),
            pltpu.VMEM((_B, _D), jnp.float32),
            pltpu.VMEM_SHARED((s_pad, _D), jnp.float32),
            pltpu.SemaphoreType.DMA,
            pltpu.SemaphoreType.DMA,
        ],
        name=name,
    )
    def spmm(table, src_hbm, dst_hbm, out, src_v, dst_v, rows_a, rows_b,
             acc, sem_a, sem_b):
        c = lax.axis_index("c")
        s = lax.axis_index("s")

        zero16 = jnp.zeros((_LANE,), jnp.float32)

        def zrow(j, _):
            for k in range(_D // _LANE):
                rows_a[j, pl.ds(k * _LANE, _LANE)] = zero16
            return 0

        lax.fori_loop(0, _B, zrow, 0)

        off = 0
        while off < s_tile:
            ch = min(_B, s_tile - off)
            pltpu.sync_copy(rows_a.at[pl.ds(0, ch)],
                            acc.at[pl.ds(s * s_tile + off, ch)])
            off += ch
        plsc.subcore_barrier()

        base = c * _NSUB * f0 + s * (f0 - c * (f0 - f1))
        nb_c = f0 - c * (f0 - f1)

        def chunk(ci, _):
            pltpu.sync_copy(src_hbm.at[pl.ds(base + ci * 8, 8)], src_v)
            pltpu.sync_copy(dst_hbm.at[pl.ds(base + ci * 8, 8)], dst_v)
            for k in range(4):
                ga = pltpu.async_copy(table.at[src_v.at[2 * k]], rows_a,
                                      sem_a)
                gb = pltpu.async_copy(table.at[src_v.at[2 * k + 1]], rows_b,
                                      sem_b)
                ga.wait()
                pltpu.sync_copy(rows_a, acc.at[dst_v.at[2 * k]], add=True)
                gb.wait()
                pltpu.sync_copy(rows_b, acc.at[dst_v.at[2 * k + 1]], add=True)
            return 0

        lax.fori_loop(0, nb_c // 8, chunk, 0)
        plsc.subcore_barrier()

        off = 0
        while off < s_tile:
            ch = min(_B, s_tile - off)
            row0 = s * s_tile + off
            pltpu.sync_copy(acc.at[pl.ds(row0, ch)], rows_a.at[pl.ds(0, ch)])
            pltpu.sync_copy(rows_a.at[pl.ds(0, ch)],
                            out.at[pl.ds(c * s_pad + row0, ch)])
            off += ch

    return spmm


def _prelu(v, a):
    return jnp.maximum(v, 0.0) + a * jnp.minimum(v, 0.0)


def _tc_pre_body(x_ref, wa_ref, ba_ref, wb_ref, a_ref, hl_ref, elsl_ref):
    a = a_ref[0, 0]
    hl = jnp.dot(x_ref[...], wa_ref[...], preferred_element_type=jnp.float32)
    hl_ref[...] = hl
    esl = _prelu(hl + ba_ref[...], a)
    elsl_ref[...] = jnp.dot(esl, wb_ref[...],
                            preferred_element_type=jnp.float32)


def _tc_pre(x, wa, ba, wb, a):
    n = x.shape[0]
    return pl.pallas_call(
        _tc_pre_body,
        out_shape=(
            jax.ShapeDtypeStruct((n, _D), jnp.float32),
            jax.ShapeDtypeStruct((n, _D), jnp.float32),
        ),
    )(x, wa, ba, wb, a)


def _tc_edge_body(acc_ref, cnt_ref, ba_ref, wb_ref, a_ref, e_ref, el_ref):
    a = a_ref[0, 0]
    cnt = cnt_ref[...]
    de_inv = jnp.where(cnt > 0, 1.0 / cnt, 0.0)
    accsum = acc_ref[...].sum(0)
    e = _prelu(accsum * de_inv + ba_ref[...], a)
    e_ref[...] = e
    el_ref[...] = jnp.dot(e, wb_ref[...], preferred_element_type=jnp.float32)


def _tc_edge(acc, cnt, ba, wb, a):
    m = acc.shape[1]
    return pl.pallas_call(
        _tc_edge_body,
        out_shape=(
            jax.ShapeDtypeStruct((m, _D), jnp.float32),
            jax.ShapeDtypeStruct((m, _D), jnp.float32),
        ),
    )(acc, cnt, ba, wb, a)


def _tc_mid_body(acc_ref, elsl_ref, cnt_ref, bb_ref, wa2_ref, ba2_ref,
                 wb2_ref, a_ref, hl2_ref, elsl2_ref):
    a = a_ref[0, 0]
    cnt = cnt_ref[...]
    dn_inv = 1.0 / (cnt + 1.0)
    n1 = (acc_ref[...].sum(0) + elsl_ref[...]) * dn_inv + bb_ref[...]
    h1 = _prelu(n1, a)
    hl2 = jnp.dot(h1, wa2_ref[...], preferred_element_type=jnp.float32)
    hl2_ref[...] = hl2
    esl2 = _prelu(hl2 + ba2_ref[...], a)
    elsl2_ref[...] = jnp.dot(esl2, wb2_ref[...],
                             preferred_element_type=jnp.float32)


def _tc_mid(acc, elsl, cnt, bb, wa2, ba2, wb2, a):
    n = acc.shape[1]
    return pl.pallas_call(
        _tc_mid_body,
        out_shape=(
            jax.ShapeDtypeStruct((n, _D), jnp.float32),
            jax.ShapeDtypeStruct((n, _D), jnp.float32),
        ),
    )(acc, elsl, cnt, bb, wa2, ba2, wb2, a)


def _tc_final_body(acc_ref, elsl_ref, cnt_ref, bb_ref, a_ref, h_ref):
    a = a_ref[0, 0]
    cnt = cnt_ref[...]
    dn_inv = 1.0 / (cnt + 1.0)
    n2 = (acc_ref[...].sum(0) + elsl_ref[...]) * dn_inv + bb_ref[...]
    h_ref[...] = _prelu(n2, a)


def _tc_final(acc, elsl, cnt, bb, a):
    n = acc.shape[1]
    return pl.pallas_call(
        _tc_final_body,
        out_shape=jax.ShapeDtypeStruct((n, _D), jnp.float32),
    )(acc, elsl, cnt, bb, a)


def kernel(x, hyperedge_index, num_nodes, num_edges, W1_n2e, b1_n2e, W1_e2n,
           b1_e2n, W2_n2e, b2_n2e, W2_e2n, b2_e2n, prelu_a):
    n, d = x.shape
    assert (n, d) == (_N, _D)
    e_inc = hyperedge_index.shape[1]

    ni = hyperedge_index[0]
    ei = hyperedge_index[1]

    nsc = 2
    f0 = 152
    f1 = 160 - f0
    fmax = max(f0, f1)
    nblk = _ceil_to(max(16 * f0 + 15 * f1 + fmax, 15 * f0 + fmax, 2560), _B)
    e_pad = nblk * _B
    pad = e_pad - e_inc
    ni_blk = jnp.concatenate(
        [ni, jnp.full((pad,), _N, jnp.int32)]).reshape(nblk, _B)
    ei_blk = jnp.concatenate(
        [ei, jnp.full((pad,), _M, jnp.int32)]).reshape(nblk, _B)

    a2 = jnp.reshape(prelu_a.astype(jnp.float32), (1, 1))
    b1a = jnp.reshape(b1_n2e, (1, _D))
    b1b = jnp.reshape(b1_e2n, (1, _D))
    b2a = jnp.reshape(b2_n2e, (1, _D))
    b2b = jnp.reshape(b2_e2n, (1, _D))

    deg_out = _make_deg(nblk)(jnp.concatenate([ni_blk, ei_blk]))
    cnt_n = deg_out[:_N, 0:1]
    cnt_e = deg_out[_NP:_NP + _MP, 0:1]

    spmm_n2e = _make_spmm(_NP, _MP, nblk, "tricon_n2e", f0, nsc)
    spmm_e2n = _make_spmm(_MP, _NP, nblk, "tricon_e2n", f0, nsc)
    zpad_n = jnp.minimum(deg_out[:_NP - _N], 0.0)

    hl1, elsl1 = _tc_pre(x, W1_n2e, b1a, W1_e2n, a2)
    hl1_pad = jnp.concatenate([hl1, zpad_n])
    acc_e1 = spmm_n2e(hl1_pad, ni_blk, ei_blk).reshape(nsc, _MP, _D)
    _e1, el1 = _tc_edge(acc_e1, cnt_e, b1a, W1_e2n, a2)
    acc_n1 = spmm_e2n(el1, ei_blk, ni_blk).reshape(nsc, _NP, _D)[:, :_N]
    hl2, elsl2 = _tc_mid(acc_n1, elsl1, cnt_n, b1b, W2_n2e, b2a, W2_e2n, a2)

    hl2_pad = jnp.concatenate([hl2, zpad_n])
    acc_e2 = spmm_n2e(hl2_pad, ni_blk, ei_blk).reshape(nsc, _MP, _D)
    e2, el2 = _tc_edge(acc_e2, cnt_e, b2a, W2_e2n, a2)
    acc_n2 = spmm_e2n(el2, ei_blk, ni_blk).reshape(nsc, _NP, _D)[:, :_N]
    h = _tc_final(acc_n2, elsl2, cnt_n, b2b, a2)

    return h, e2[:_M]


# speedup vs baseline: 1.0138x; 1.0138x over previous
import functools

import jax
import jax.numpy as jnp
from jax import lax
from jax.experimental import pallas as pl
from jax.experimental.pallas import tpu as pltpu
from jax.experimental.pallas import tpu_sc as plsc

_N = 10000
_M = 5000
_D = 128
_NP = 10112
_MP = 5120
_NCORE = 2
_NSUB = 16
_NTILE = _NCORE * _NSUB
_LANE = 16
_B = 128


def _ceil_to(v, m):
    return (v + m - 1) // m * m


def _mesh(nc=_NCORE):
    return plsc.VectorSubcoreMesh(core_axis_name="c", subcore_axis_name="s",
                                  num_cores=nc)


def _make_deg(nblk):
    nb_sub = nblk // _NSUB
    n_tile = _NP // _NSUB

    @functools.partial(
        pl.kernel,
        out_type=jax.ShapeDtypeStruct((_NCORE * _NP, _D), jnp.float32),
        mesh=_mesh(),
        scratch_types=[
            pltpu.VMEM((nb_sub, _B), jnp.int32),
            pltpu.VMEM((_B, _D), jnp.float32),
            pltpu.VMEM_SHARED((_NP, _D), jnp.float32),
        ],
        name="tricon_deg",
    )
    def deg(idx_hbm, out, idx_v, ones_v, hist):
        c = lax.axis_index("c")
        s = lax.axis_index("s")

        zero16 = jnp.zeros((_LANE,), jnp.float32)

        def zrow(j, _):
            for k in range(_D // _LANE):
                ones_v[j, pl.ds(k * _LANE, _LANE)] = zero16
            return 0

        lax.fori_loop(0, _B, zrow, 0)

        off = 0
        while off < n_tile:
            ch = min(_B, n_tile - off)
            pltpu.sync_copy(ones_v.at[pl.ds(0, ch)],
                            hist.at[pl.ds(s * n_tile + off, ch)])
            off += ch

        one16 = jnp.full((_LANE,), 1.0, jnp.float32)

        def orow(j, _):
            for k in range(_D // _LANE):
                ones_v[j, pl.ds(k * _LANE, _LANE)] = one16
            return 0

        lax.fori_loop(0, _B, orow, 0)
        plsc.subcore_barrier()

        pltpu.sync_copy(idx_hbm.at[pl.ds(c * nblk + s * nb_sub, nb_sub)],
                        idx_v)

        def body(j, _):
            pltpu.sync_copy(ones_v, hist.at[idx_v.at[j]], add=True)
            return 0

        lax.fori_loop(0, nb_sub, body, 0)
        plsc.subcore_barrier()

        off = 0
        while off < n_tile:
            ch = min(_B, n_tile - off)
            row0 = s * n_tile + off
            pltpu.sync_copy(hist.at[pl.ds(row0, ch)],
                            ones_v.at[pl.ds(0, ch)])
            pltpu.sync_copy(ones_v.at[pl.ds(0, ch)],
                            out.at[pl.ds(c * _NP + row0, ch)])
            off += ch

    return deg


def _make_spmm(r_pad, s_pad, nblk, name, f0, ncores=_NCORE):
    nb_pair = 160
    f1 = nb_pair - f0
    fmax = max(f0, f1)
    s_tile = s_pad // _NSUB

    @functools.partial(
        pl.kernel,
        out_type=jax.ShapeDtypeStruct((ncores * s_pad, _D), jnp.float32),
        mesh=_mesh(ncores),
        scratch_types=[
            pltpu.VMEM((8, _B), jnp.int32),
            pltpu.VMEM((8, _B), jnp.int32),
            pltpu.VMEM((_B, _D), jnp.float32),
            pltpu.VMEM((_B, _D), jnp.float32),
            pltpu.VMEM_SHARED((s_pad, _D), jnp.float32),
            pltpu.SemaphoreType.DMA,
            pltpu.SemaphoreType.DMA,
        ],
        name=name,
    )
    def spmm(table, src_hbm, dst_hbm, out, src_v, dst_v, rows_a, rows_b,
             acc, sem_a, sem_b):
        c = lax.axis_index("c")
        s = lax.axis_index("s")

        zero16 = jnp.zeros((_LANE,), jnp.float32)

        def zrow(j, _):
            for k in range(_D // _LANE):
                rows_a[j, pl.ds(k * _LANE, _LANE)] = zero16
            return 0

        lax.fori_loop(0, _B, zrow, 0)

        off = 0
        while off < s_tile:
            ch = min(_B, s_tile - off)
            pltpu.sync_copy(rows_a.at[pl.ds(0, ch)],
                            acc.at[pl.ds(s * s_tile + off, ch)])
            off += ch
        plsc.subcore_barrier()

        base = c * _NSUB * f0 + s * (f0 - c * (f0 - f1))
        nb_c = f0 - c * (f0 - f1)

        def chunk(ci, _):
            pltpu.sync_copy(src_hbm.at[pl.ds(base + ci * 8, 8)], src_v)
            pltpu.sync_copy(dst_hbm.at[pl.ds(base + ci * 8, 8)], dst_v)
            for k in range(4):
                ga = pltpu.async_copy(table.at[src_v.at[2 * k]], rows_a,
                                      sem_a)
                gb = pltpu.async_copy(table.at[src_v.at[2 * k + 1]], rows_b,
                                      sem_b)
                ga.wait()
                pltpu.sync_copy(rows_a, acc.at[dst_v.at[2 * k]], add=True)
                gb.wait()
                pltpu.sync_copy(rows_b, acc.at[dst_v.at[2 * k + 1]], add=True)
            return 0

        lax.fori_loop(0, nb_c // 8, chunk, 0)
        plsc.subcore_barrier()

        off = 0
        while off < s_tile:
            ch = min(_B, s_tile - off)
            row0 = s * s_tile + off
            pltpu.sync_copy(acc.at[pl.ds(row0, ch)], rows_a.at[pl.ds(0, ch)])
            pltpu.sync_copy(rows_a.at[pl.ds(0, ch)],
                            out.at[pl.ds(c * s_pad + row0, ch)])
            off += ch

    return spmm


def _prelu(v, a):
    return jnp.maximum(v, 0.0) + a * jnp.minimum(v, 0.0)


def _tc_pre_body(x_ref, wa_ref, ba_ref, wb_ref, a_ref, hl_ref, elsl_ref):
    a = a_ref[0, 0]
    hl = jnp.dot(x_ref[...], wa_ref[...], preferred_element_type=jnp.float32)
    hl_ref[...] = hl
    esl = _prelu(hl + ba_ref[...], a)
    elsl_ref[...] = jnp.dot(esl, wb_ref[...],
                            preferred_element_type=jnp.float32)


def _tc_pre(x, wa, ba, wb, a):
    n = x.shape[0]
    return pl.pallas_call(
        _tc_pre_body,
        out_shape=(
            jax.ShapeDtypeStruct((n, _D), jnp.float32),
            jax.ShapeDtypeStruct((n, _D), jnp.float32),
        ),
    )(x, wa, ba, wb, a)


def _tc_edge_body(acc_ref, cnt_ref, ba_ref, wb_ref, a_ref, e_ref, el_ref):
    a = a_ref[0, 0]
    cnt = cnt_ref[...]
    de_inv = jnp.where(cnt > 0, 1.0 / cnt, 0.0)
    accsum = acc_ref[...].sum(0)
    e = _prelu(accsum * de_inv + ba_ref[...], a)
    e_ref[...] = e
    el_ref[...] = jnp.dot(e, wb_ref[...], preferred_element_type=jnp.float32)


def _tc_edge(acc, cnt, ba, wb, a):
    m = acc.shape[1]
    return pl.pallas_call(
        _tc_edge_body,
        out_shape=(
            jax.ShapeDtypeStruct((m, _D), jnp.float32),
            jax.ShapeDtypeStruct((m, _D), jnp.float32),
        ),
    )(acc, cnt, ba, wb, a)


def _tc_mid_body(acc_ref, elsl_ref, cnt_ref, bb_ref, wa2_ref, ba2_ref,
                 wb2_ref, a_ref, hl2_ref, elsl2_ref):
    a = a_ref[0, 0]
    cnt = cnt_ref[...]
    dn_inv = 1.0 / (cnt + 1.0)
    n1 = (acc_ref[...].sum(0) + elsl_ref[...]) * dn_inv + bb_ref[...]
    h1 = _prelu(n1, a)
    hl2 = jnp.dot(h1, wa2_ref[...], preferred_element_type=jnp.float32)
    hl2_ref[...] = hl2
    esl2 = _prelu(hl2 + ba2_ref[...], a)
    elsl2_ref[...] = jnp.dot(esl2, wb2_ref[...],
                             preferred_element_type=jnp.float32)


def _tc_mid(acc, elsl, cnt, bb, wa2, ba2, wb2, a):
    n = acc.shape[1]
    return pl.pallas_call(
        _tc_mid_body,
        out_shape=(
            jax.ShapeDtypeStruct((n, _D), jnp.float32),
            jax.ShapeDtypeStruct((n, _D), jnp.float32),
        ),
    )(acc, elsl, cnt, bb, wa2, ba2, wb2, a)


def _tc_final_body(acc_ref, elsl_ref, cnt_ref, bb_ref, a_ref, h_ref):
    a = a_ref[0, 0]
    cnt = cnt_ref[...]
    dn_inv = 1.0 / (cnt + 1.0)
    n2 = (acc_ref[...].sum(0) + elsl_ref[...]) * dn_inv + bb_ref[...]
    h_ref[...] = _prelu(n2, a)


def _tc_final(acc, elsl, cnt, bb, a):
    n = acc.shape[1]
    return pl.pallas_call(
        _tc_final_body,
        out_shape=jax.ShapeDtypeStruct((n, _D), jnp.float32),
    )(acc, elsl, cnt, bb, a)


def kernel(x, hyperedge_index, num_nodes, num_edges, W1_n2e, b1_n2e, W1_e2n,
           b1_e2n, W2_n2e, b2_n2e, W2_e2n, b2_e2n, prelu_a):
    n, d = x.shape
    assert (n, d) == (_N, _D)
    e_inc = hyperedge_index.shape[1]

    ni = hyperedge_index[0]
    ei = hyperedge_index[1]

    nsc = 2
    f0 = 144
    f1 = 160 - f0
    fmax = max(f0, f1)
    nblk = _ceil_to(max(16 * f0 + 15 * f1 + fmax, 15 * f0 + fmax, 2560), _B)
    e_pad = nblk * _B
    pad = e_pad - e_inc
    ni_blk = jnp.concatenate(
        [ni, jnp.full((pad,), _N, jnp.int32)]).reshape(nblk, _B)
    ei_blk = jnp.concatenate(
        [ei, jnp.full((pad,), _M, jnp.int32)]).reshape(nblk, _B)

    a2 = jnp.reshape(prelu_a.astype(jnp.float32), (1, 1))
    b1a = jnp.reshape(b1_n2e, (1, _D))
    b1b = jnp.reshape(b1_e2n, (1, _D))
    b2a = jnp.reshape(b2_n2e, (1, _D))
    b2b = jnp.reshape(b2_e2n, (1, _D))

    deg_out = _make_deg(nblk)(jnp.concatenate([ni_blk, ei_blk]))
    cnt_n = deg_out[:_N, 0:1]
    cnt_e = deg_out[_NP:_NP + _MP, 0:1]

    spmm_n2e = _make_spmm(_NP, _MP, nblk, "tricon_n2e", f0, nsc)
    spmm_e2n = _make_spmm(_MP, _NP, nblk, "tricon_e2n", f0, nsc)
    zpad_n = jnp.minimum(deg_out[:_NP - _N], 0.0)

    hl1, elsl1 = _tc_pre(x, W1_n2e, b1a, W1_e2n, a2)
    hl1_pad = jnp.concatenate([hl1, zpad_n])
    acc_e1 = spmm_n2e(hl1_pad, ni_blk, ei_blk).reshape(nsc, _MP, _D)
    _e1, el1 = _tc_edge(acc_e1, cnt_e, b1a, W1_e2n, a2)
    acc_n1 = spmm_e2n(el1, ei_blk, ni_blk).reshape(nsc, _NP, _D)[:, :_N]
    hl2, elsl2 = _tc_mid(acc_n1, elsl1, cnt_n, b1b, W2_n2e, b2a, W2_e2n, a2)

    hl2_pad = jnp.concatenate([hl2, zpad_n])
    acc_e2 = spmm_n2e(hl2_pad, ni_blk, ei_blk).reshape(nsc, _MP, _D)
    e2, el2 = _tc_edge(acc_e2, cnt_e, b2a, W2_e2n, a2)
    acc_n2 = spmm_e2n(el2, ei_blk, ni_blk).reshape(nsc, _NP, _D)[:, :_N]
    h = _tc_final(acc_n2, elsl2, cnt_n, b2b, a2)

    return h, e2[:_M]
